# Initial kernel scaffold; baseline (speedup 1.0000x reference)
import jax
import jax.numpy as jnp
from jax.experimental import pallas as pl


def kernel(gene, genes):
    raise NotImplementedError("write your pallas kernel here")


# speedup vs baseline: 264.4919x; 264.4919x over previous
import functools

import jax
import jax.numpy as jnp
from jax import lax
from jax.experimental import pallas as pl
from jax.experimental.pallas import tpu as pltpu
from jax.experimental.pallas import tpu_sc as plsc

B = 16384
NV = 100
NG = 1000
NW = 32
RW = B // NW
CH = 128
NCH = RW // CH
L = 16


def _sc_body(gene_hbm, table_hbm, out_hbm, table_v, gidx_v, out_v):
    wid = lax.axis_index("s") * 2 + lax.axis_index("c")
    base_row = wid * RW

    pltpu.sync_copy(table_hbm, table_v)

    lane_rows = lax.iota(jnp.int32, L) * NV

    for c in range(NCH):
        chunk_row = base_row + c * CH
        pltpu.sync_copy(gene_hbm.at[pl.ds(chunk_row * NV, CH * NV)], gidx_v)
        for i0 in range(CH // L):
            group_base = i0 * L * NV

            def body(v, acc):
                gsel = lane_rows + (group_base + v)
                g = plsc.load_gather(gidx_v, [gsel])
                val = plsc.load_gather(table_v, [g + v * NG])
                return acc + val

            acc = lax.fori_loop(0, NV, body, jnp.zeros((L,), jnp.float32))
            out_v[pl.ds(c * CH + i0 * L, L)] = acc

    pltpu.sync_copy(out_v, out_hbm.at[pl.ds(base_row, RW)])


@jax.jit
def kernel(gene, genes):
    gene_flat = gene.reshape(-1).astype(jnp.int32)
    table_flat = genes.reshape(-1).astype(jnp.float32)

    sc_call = functools.partial(
        pl.kernel,
        mesh=plsc.VectorSubcoreMesh(core_axis_name="c", subcore_axis_name="s"),
        out_type=jax.ShapeDtypeStruct((B,), jnp.float32),
        scratch_types=[
            pltpu.VMEM((NV * NG,), jnp.float32),
            pltpu.VMEM((CH * NV,), jnp.int32),
            pltpu.VMEM((RW,), jnp.float32),
        ],
        compiler_params=pltpu.CompilerParams(needs_layout_passes=False),
    )(_sc_body)

    fit = sc_call(gene_flat, table_flat)
    return fit.reshape(B, 1)


# speedup vs baseline: 302.5467x; 1.1439x over previous
import functools

import jax
import jax.numpy as jnp
from jax import lax
from jax.experimental import pallas as pl
from jax.experimental.pallas import tpu as pltpu
from jax.experimental.pallas import tpu_sc as plsc

B = 16384
NV = 100
NG = 1000
NW = 32
RW = B // NW
CH = 128
NCH = RW // CH
L = 16


def _sc_body(gene_hbm, table_hbm, out_hbm,
             table_v, g0_v, g1_v, out_v, sem_t, sem0, sem1):
    wid = lax.axis_index("s") * 2 + lax.axis_index("c")
    base_row = wid * RW

    tbl_cp = pltpu.make_async_copy(table_hbm, table_v, sem_t)
    tbl_cp.start()

    bufs = (g0_v, g1_v)
    sems = (sem0, sem1)

    def gene_copy(c):
        return pltpu.make_async_copy(
            gene_hbm.at[pl.ds((base_row + c * CH) * NV, CH * NV)],
            bufs[c % 2], sems[c % 2])

    cp = gene_copy(0)
    cp.start()
    tbl_cp.wait()

    lane_rows = lax.iota(jnp.int32, L) * NV

    for c in range(NCH):
        cp.wait()
        if c + 1 < NCH:
            cp = gene_copy(c + 1)
            cp.start()
        gidx_v = bufs[c % 2]
        for i0 in range(CH // L):
            group_base = i0 * L * NV

            def body(v, acc):
                gsel = lane_rows + (group_base + v)
                g = plsc.load_gather(gidx_v, [gsel])
                val = plsc.load_gather(table_v, [g + v * NG])
                return acc + val

            acc = lax.fori_loop(0, NV, body, jnp.zeros((L,), jnp.float32),
                                unroll=10)
            out_v[pl.ds(c * CH + i0 * L, L)] = acc

    pltpu.sync_copy(out_v, out_hbm.at[pl.ds(base_row, RW)])


@jax.jit
def kernel(gene, genes):
    gene_flat = gene.reshape(-1).astype(jnp.int32)
    table_flat = genes.reshape(-1).astype(jnp.float32)

    sc_call = functools.partial(
        pl.kernel,
        mesh=plsc.VectorSubcoreMesh(core_axis_name="c", subcore_axis_name="s"),
        out_type=jax.ShapeDtypeStruct((B,), jnp.float32),
        scratch_types=[
            pltpu.VMEM((NV * NG,), jnp.float32),
            pltpu.VMEM((CH * NV,), jnp.int32),
            pltpu.VMEM((CH * NV,), jnp.int32),
            pltpu.VMEM((RW,), jnp.float32),
            pltpu.SemaphoreType.DMA,
            pltpu.SemaphoreType.DMA,
            pltpu.SemaphoreType.DMA,
        ],
        compiler_params=pltpu.CompilerParams(needs_layout_passes=False),
    )(_sc_body)

    fit = sc_call(gene_flat, table_flat)
    return fit.reshape(B, 1)


# speedup vs baseline: 303.6014x; 1.0035x over previous
import functools

import jax
import jax.numpy as jnp
from jax import lax
from jax.experimental import pallas as pl
from jax.experimental.pallas import tpu as pltpu
from jax.experimental.pallas import tpu_sc as plsc

B = 16384
NV = 100
NVP = 128
NG = 1000
NW = 32
RW = B // NW
CH = 64
NCH = RW // CH
L = 16
NK = 7


def _sc_body(gene_hbm, table_hbm, out_hbm,
             table_v, g0_v, g1_v, out_v, sem_t, sem0, sem1):
    wid = lax.axis_index("s") * 2 + lax.axis_index("c")
    base_row = wid * RW

    tbl_cp = pltpu.make_async_copy(table_hbm, table_v, sem_t)
    tbl_cp.start()

    bufs = (g0_v, g1_v)
    sems = (sem0, sem1)

    def gene_copy(c):
        return pltpu.make_async_copy(
            gene_hbm.at[pl.ds((base_row + c * CH) * NVP, CH * NVP)],
            bufs[c % 2], sems[c % 2])

    cp = gene_copy(0)
    cp.start()
    tbl_cp.wait()

    lanes = lax.iota(jnp.int32, L)
    tail_valid = lanes < (NV - (NK - 1) * L)
    voffs = [(lanes + k * L) * NG for k in range(NK - 1)]
    voffs.append(jnp.where(tail_valid, (lanes + (NK - 1) * L) * NG, 0))
    fzero = jnp.zeros((L,), jnp.float32)

    for c in range(NCH):
        cp.wait()
        if c + 1 < NCH:
            cp = gene_copy(c + 1)
            cp.start()
        gbuf = bufs[c % 2]

        def rows_body(i, _):
            rbase = i * L
            sums = fzero
            for j in range(L):
                roff = (rbase + j) * NVP
                acc = fzero
                for k in range(NK):
                    g = gbuf[pl.ds(roff + k * L, L)]
                    val = plsc.load_gather(table_v, [g + voffs[k]])
                    if k == NK - 1:
                        val = jnp.where(tail_valid, val, 0.0)
                    acc = acc + val
                sums = jnp.where(lanes == j, jnp.sum(acc), sums)
            out_v[pl.ds(c * CH + rbase, L)] = sums
            return 0

        lax.fori_loop(0, CH // L, rows_body, 0)

    pltpu.sync_copy(out_v, out_hbm.at[pl.ds(base_row, RW)])


@jax.jit
def kernel(gene, genes):
    gene_p = jnp.pad(gene.astype(jnp.int32), ((0, 0), (0, NVP - NV)))
    table_flat = genes.reshape(-1).astype(jnp.float32)

    sc_call = functools.partial(
        pl.kernel,
        mesh=plsc.VectorSubcoreMesh(core_axis_name="c", subcore_axis_name="s"),
        out_type=jax.ShapeDtypeStruct((B,), jnp.float32),
        scratch_types=[
            pltpu.VMEM((NV * NG,), jnp.float32),
            pltpu.VMEM((CH * NVP,), jnp.int32),
            pltpu.VMEM((CH * NVP,), jnp.int32),
            pltpu.VMEM((RW,), jnp.float32),
            pltpu.SemaphoreType.DMA,
            pltpu.SemaphoreType.DMA,
            pltpu.SemaphoreType.DMA,
        ],
        compiler_params=pltpu.CompilerParams(needs_layout_passes=False),
    )(_sc_body)

    fit = sc_call(gene_p.reshape(-1), table_flat)
    return fit.reshape(B, 1)


# speedup vs baseline: 387.8945x; 1.2776x over previous
import functools

import jax
import jax.numpy as jnp
from jax import lax
from jax.experimental import pallas as pl
from jax.experimental.pallas import tpu as pltpu
from jax.experimental.pallas import tpu_sc as plsc

B = 16384
NV = 100
NG = 1000
NW = 32
RW = B // NW
CH = 64
NCH = RW // CH
L = 16
NF = 6
TB = NV - L


def _sc_body(gene_hbm, table_hbm, out_hbm,
             table_v, g0_v, g1_v, out_v, sem_t, sem0, sem1):
    wid = lax.axis_index("s") * 2 + lax.axis_index("c")
    base_row = wid * RW

    tbl_cp = pltpu.make_async_copy(table_hbm, table_v, sem_t)
    tbl_cp.start()

    bufs = (g0_v, g1_v)
    sems = (sem0, sem1)

    def gene_copy(c):
        return pltpu.make_async_copy(
            gene_hbm.at[pl.ds(base_row + c * CH, CH), :],
            bufs[c % 2], sems[c % 2])

    cp = gene_copy(0)
    cp.start()
    tbl_cp.wait()

    lanes = lax.iota(jnp.int32, L)
    tail_fresh = lanes >= (L - (NV - NF * L))
    voffs = [(lanes + k * L) * NG for k in range(NF)]
    voffs_tail = (lanes + TB) * NG
    fzero = jnp.zeros((L,), jnp.float32)

    for c in range(NCH):
        cp.wait()
        if c + 1 < NCH:
            cp = gene_copy(c + 1)
            cp.start()
        gbuf = bufs[c % 2]

        def rows_body(i, _):
            rbase = i * L
            sums = fzero
            for j in range(L):
                r = rbase + j
                acc = fzero
                for k in range(NF):
                    g = gbuf[r, pl.ds(k * L, L)]
                    acc = acc + plsc.load_gather(table_v, [g + voffs[k]])
                rvec = jnp.full((L,), r, jnp.int32)
                gt = plsc.load_gather(gbuf, [rvec, TB + lanes])
                vt = plsc.load_gather(table_v, [gt + voffs_tail])
                acc = acc + jnp.where(tail_fresh, vt, 0.0)
                sums = jnp.where(lanes == j, jnp.sum(acc), sums)
            out_v[pl.ds(c * CH + rbase, L)] = sums
            return 0

        lax.fori_loop(0, CH // L, rows_body, 0)

    pltpu.sync_copy(out_v, out_hbm.at[pl.ds(base_row, RW)])


@jax.jit
def kernel(gene, genes):
    table_flat = genes.reshape(-1).astype(jnp.float32)

    sc_call = functools.partial(
        pl.kernel,
        mesh=plsc.VectorSubcoreMesh(core_axis_name="c", subcore_axis_name="s"),
        out_type=jax.ShapeDtypeStruct((B,), jnp.float32),
        scratch_types=[
            pltpu.VMEM((NV * NG,), jnp.float32),
            pltpu.VMEM((CH, NV), jnp.int32),
            pltpu.VMEM((CH, NV), jnp.int32),
            pltpu.VMEM((RW,), jnp.float32),
            pltpu.SemaphoreType.DMA,
            pltpu.SemaphoreType.DMA,
            pltpu.SemaphoreType.DMA,
        ],
        compiler_params=pltpu.CompilerParams(needs_layout_passes=False),
    )(_sc_body)

    fit = sc_call(gene.astype(jnp.int32), table_flat)
    return fit.reshape(B, 1)
